# Initial kernel scaffold; baseline (speedup 1.0000x reference)
import jax
import jax.numpy as jnp
from jax.experimental import pallas as pl


def kernel(x, y):
    raise NotImplementedError("write your pallas kernel here")


# speedup vs baseline: 3.2514x; 3.2514x over previous
import jax
import jax.numpy as jnp
from jax.experimental import pallas as pl
from jax.experimental.pallas import tpu as pltpu

_B = 4096
_C = 1000
_R = 512
_G = _B // _R


def _rce_kernel(x_ref, y_ref, out_ref, colsum_acc, counts_acc):
    i = pl.program_id(0)

    @pl.when(i == 0)
    def _init():
        colsum_acc[...] = jnp.zeros_like(colsum_acc)
        counts_acc[...] = jnp.zeros_like(counts_acc)

    xb = x_ref[...]
    colsum_acc[...] += jnp.sum(xb, axis=0, keepdims=True)

    yb = y_ref[0]
    yv = yb.reshape(_R, 1)
    classes = jax.lax.broadcasted_iota(jnp.int32, (1, _C), 1)
    onehot = (yv == classes).astype(jnp.float32)
    counts_acc[...] += jnp.sum(onehot, axis=0, keepdims=True)

    @pl.when(i == _G - 1)
    def _final():
        s = jnp.sum(colsum_acc[...] * counts_acc[...], keepdims=True)
        out_ref[...] = 6.0 - (6.0 / (_B * _B)) * s


def kernel(x, y):
    y3 = y.astype(jnp.int32).reshape(_G, 1, _R)
    out = pl.pallas_call(
        _rce_kernel,
        grid=(_G,),
        in_specs=[
            pl.BlockSpec((_R, _C), lambda i: (i, 0)),
            pl.BlockSpec((1, 1, _R), lambda i: (i, 0, 0)),
        ],
        out_specs=pl.BlockSpec((1, 1), lambda i: (0, 0)),
        out_shape=jax.ShapeDtypeStruct((1, 1), jnp.float32),
        scratch_shapes=[
            pltpu.VMEM((1, _C), jnp.float32),
            pltpu.VMEM((1, _C), jnp.float32),
        ],
    )(x, y3)
    return jnp.reshape(out, ())


# speedup vs baseline: 3.4588x; 1.0638x over previous
import jax
import jax.numpy as jnp
from jax.experimental import pallas as pl
from jax.experimental.pallas import tpu as pltpu

_B = 4096
_C = 1000
_R = 1024
_G = _B // _R


def _rce_kernel(x_ref, y_ref, out_ref, colsum_acc, counts_acc):
    i = pl.program_id(0)

    @pl.when(i == 0)
    def _init():
        colsum_acc[...] = jnp.zeros_like(colsum_acc)
        counts_acc[...] = jnp.zeros_like(counts_acc)

    xb = x_ref[...]
    colsum_acc[...] += jnp.sum(xb, axis=0, keepdims=True)

    yb = y_ref[0]
    yv = yb.reshape(_R, 1)
    classes = jax.lax.broadcasted_iota(jnp.int32, (1, _C), 1)
    onehot = (yv == classes).astype(jnp.float32)
    counts_acc[...] += jnp.sum(onehot, axis=0, keepdims=True)

    @pl.when(i == _G - 1)
    def _final():
        s = jnp.sum(colsum_acc[...] * counts_acc[...], keepdims=True)
        out_ref[...] = 6.0 - (6.0 / (_B * _B)) * s


def kernel(x, y):
    y3 = y.astype(jnp.int32).reshape(_G, 1, _R)
    out = pl.pallas_call(
        _rce_kernel,
        grid=(_G,),
        in_specs=[
            pl.BlockSpec((_R, _C), lambda i: (i, 0)),
            pl.BlockSpec((1, 1, _R), lambda i: (i, 0, 0)),
        ],
        out_specs=pl.BlockSpec((1, 1), lambda i: (0, 0)),
        out_shape=jax.ShapeDtypeStruct((1, 1), jnp.float32),
        scratch_shapes=[
            pltpu.VMEM((1, _C), jnp.float32),
            pltpu.VMEM((1, _C), jnp.float32),
        ],
    )(x, y3)
    return jnp.reshape(out, ())
